# Initial kernel scaffold; baseline (speedup 1.0000x reference)
import jax
import jax.numpy as jnp
from jax.experimental import pallas as pl


def kernel(x, conv_w, conv_b):
    raise NotImplementedError("write your pallas kernel here")


# speedup vs baseline: 25.8556x; 25.8556x over previous
import functools

import jax
import jax.numpy as jnp
from jax import lax
from jax.experimental import pallas as pl
from jax.experimental.pallas import tpu as pltpu
from jax.experimental.pallas import tpu_sc as plsc

H = 12
T = 2048
DK = 64
K = 4
TM = 256
NB = T // TM

NC = 2
NS = 16
NW = NC * NS
POS = H * T
PPW = POS // NW
CH = 128
NCHUNK = PPW // CH


def _sim_topk_body(xfull_ref, xblk_ref, wt_ref, fidx_ref, y_ref):
    h = pl.program_id(0)
    xfull = xfull_ref[0]
    xblk = xblk_ref[0]
    nf = jnp.sqrt(jnp.sum(xfull * xfull, axis=1, keepdims=True))
    xn_full = xfull / jnp.maximum(nf, 1e-12)
    nb = jnp.sqrt(jnp.sum(xblk * xblk, axis=1, keepdims=True))
    xn_blk = xblk / jnp.maximum(nb, 1e-12)
    s = lax.dot_general(xn_blk, xn_full, (((1,), (1,)), ((), ())),
                        preferred_element_type=jnp.float32)
    s = jnp.maximum(s, 0.0)
    iota = lax.broadcasted_iota(jnp.int32, (TM, T), 1)
    cols = []
    for _ in range(K):
        m = jnp.max(s, axis=1, keepdims=True)
        ii = jnp.min(jnp.where(s == m, iota, T), axis=1, keepdims=True)
        cols.append(ii)
        s = jnp.where(iota == ii, -1.0, s)
    idx = jnp.concatenate(cols, axis=1)
    jj = lax.broadcasted_iota(jnp.int32, (TM, K), 1)
    fidx_ref[0] = (h * T + idx) * K + jj
    ys = [lax.dot_general(xblk, wt_ref[j], (((1,), (0,)), ((), ())),
                          preferred_element_type=jnp.float32)
          for j in range(K)]
    y_ref[0] = jnp.concatenate(ys, axis=1)


def _sim_topk(xh, wt):
    return pl.pallas_call(
        _sim_topk_body,
        grid=(H, NB),
        in_specs=[
            pl.BlockSpec((1, T, DK), lambda h, i: (h, 0, 0)),
            pl.BlockSpec((1, TM, DK), lambda h, i: (h, i, 0)),
            pl.BlockSpec((K, DK, DK), lambda h, i: (0, 0, 0)),
        ],
        out_specs=[
            pl.BlockSpec((1, TM, K), lambda h, i: (h, i, 0)),
            pl.BlockSpec((1, TM, K * DK), lambda h, i: (h, i, 0)),
        ],
        out_shape=[
            jax.ShapeDtypeStruct((H, T, K), jnp.int32),
            jax.ShapeDtypeStruct((H, T, K * DK), jnp.float32),
        ],
    )(xh, xh, wt)


def _gather_sum_body(table_hbm, fidx_hbm, bias_hbm, out_hbm,
                     idx_v, rows_v, out_v, bias_v, sem):
    cid = lax.axis_index("c")
    sid = lax.axis_index("s")
    wid = sid * NC + cid
    pltpu.sync_copy(bias_hbm, bias_v)

    def chunk_body(ci, carry):
        gchunk = wid * NCHUNK + ci
        base = gchunk * CH
        pltpu.sync_copy(fidx_hbm.at[gchunk], idx_v)
        copies = [
            pltpu.async_copy(table_hbm.at[idx_v.at[g]],
                             rows_v.at[pl.ds(g * CH, CH)], sem)
            for g in range(K)
        ]
        for cp in copies:
            cp.wait()

        def pos_body(i, c2):
            for c4 in range(DK // 16):
                sl = pl.ds(c4 * 16, 16)
                acc = bias_v[sl]
                for j in range(K):
                    acc = acc + rows_v[i * K + j, sl]
                out_v[i, sl] = acc
            return c2

        lax.fori_loop(0, CH, pos_body, 0)
        pltpu.sync_copy(out_v, out_hbm.at[pl.ds(base, CH)])
        return carry

    lax.fori_loop(0, NCHUNK, chunk_body, 0)


@functools.cache
def _gather_sum():
    mesh = plsc.VectorSubcoreMesh(core_axis_name="c", subcore_axis_name="s")
    return pl.kernel(
        _gather_sum_body,
        mesh=mesh,
        out_type=jax.ShapeDtypeStruct((POS, DK), jnp.float32),
        scratch_types=[
            pltpu.VMEM((K, CH), jnp.int32),
            pltpu.VMEM((K * CH, DK), jnp.float32),
            pltpu.VMEM((CH, DK), jnp.float32),
            pltpu.VMEM((DK,), jnp.float32),
            pltpu.SemaphoreType.DMA,
        ],
        compiler_params=pltpu.CompilerParams(use_tc_tiling_on_sc=False),
    )


def kernel(x, conv_w, conv_b):
    xh = x.reshape(T, H, DK).transpose(1, 0, 2)
    wt = conv_w.transpose(2, 1, 0)
    fidx, y = _sim_topk(xh, wt)
    table = y.reshape(POS * K, DK)
    fidx_r = fidx.reshape(NW * NCHUNK, K, CH)
    out_sc = _gather_sum()(table, fidx_r, conv_b)
    y2 = out_sc.reshape(1, H, DK, T)
    return y2.transpose(0, 1, 3, 2).transpose(0, 2, 1, 3).reshape(1, T, H * DK)


# speedup vs baseline: 27.9145x; 1.0796x over previous
import functools

import jax
import jax.numpy as jnp
from jax import lax
from jax.experimental import pallas as pl
from jax.experimental.pallas import tpu as pltpu
from jax.experimental.pallas import tpu_sc as plsc

H = 12
T = 2048
DK = 64
K = 4
TM = 256
NB = T // TM

NC = 2
NS = 16
NW = NC * NS
POS = H * T
PPW = POS // NW
CH = 128
NCHUNK = PPW // CH


def _sim_topk_body(xfull_ref, xblk_ref, wt_ref, fidx_ref, y_ref, xn_scr):
    h = pl.program_id(0)
    i = pl.program_id(1)
    xblk = xblk_ref[0]

    @pl.when(i == 0)
    def _():
        xfull = xfull_ref[0]
        nf = jnp.sqrt(jnp.sum(xfull * xfull, axis=1, keepdims=True))
        xn_scr[...] = xfull / jnp.maximum(nf, 1e-12)

    xn_full = xn_scr[...]
    xn_blk = xn_scr[pl.ds(i * TM, TM), :]
    s = lax.dot_general(xn_blk, xn_full, (((1,), (1,)), ((), ())),
                        preferred_element_type=jnp.float32)
    s = jnp.maximum(s, 0.0)
    iota = lax.broadcasted_iota(jnp.int32, (TM, T), 1)
    cols = []
    for _ in range(K):
        m = jnp.max(s, axis=1, keepdims=True)
        ii = jnp.min(jnp.where(s == m, iota, T), axis=1, keepdims=True)
        cols.append(ii)
        s = jnp.where(iota == ii, -1.0, s)
    idx = jnp.concatenate(cols, axis=1)
    jj = lax.broadcasted_iota(jnp.int32, (TM, K), 1)
    fidx_ref[0] = (h * T + idx) * K + jj
    ys = [lax.dot_general(xblk, wt_ref[j], (((1,), (0,)), ((), ())),
                          preferred_element_type=jnp.float32)
          for j in range(K)]
    y_ref[0] = jnp.concatenate(ys, axis=1)


def _sim_topk(xh, wt):
    return pl.pallas_call(
        _sim_topk_body,
        grid=(H, NB),
        in_specs=[
            pl.BlockSpec((1, T, DK), lambda h, i: (h, 0, 0)),
            pl.BlockSpec((1, TM, DK), lambda h, i: (h, i, 0)),
            pl.BlockSpec((K, DK, DK), lambda h, i: (0, 0, 0)),
        ],
        out_specs=[
            pl.BlockSpec((1, TM, K), lambda h, i: (h, i, 0)),
            pl.BlockSpec((1, TM, K * DK), lambda h, i: (h, i, 0)),
        ],
        out_shape=[
            jax.ShapeDtypeStruct((H, T, K), jnp.int32),
            jax.ShapeDtypeStruct((H, T, K * DK), jnp.float32),
        ],
        scratch_shapes=[pltpu.VMEM((T, DK), jnp.float32)],
    )(xh, xh, wt)


def _gather_sum_body(table_hbm, fidx_hbm, bias_hbm, out_hbm,
                     idx_v, rows_v, out_v, bias_v, sem):
    cid = lax.axis_index("c")
    sid = lax.axis_index("s")
    wid = sid * NC + cid
    pltpu.sync_copy(bias_hbm, bias_v)

    def chunk_body(ci, carry):
        gchunk = wid * NCHUNK + ci
        base = gchunk * CH
        pltpu.sync_copy(fidx_hbm.at[gchunk], idx_v)
        copies = [
            pltpu.async_copy(table_hbm.at[idx_v.at[g]],
                             rows_v.at[pl.ds(g * CH, CH)], sem)
            for g in range(K)
        ]
        for cp in copies:
            cp.wait()

        def pos_body(i, c2):
            for c4 in range(DK // 16):
                sl = pl.ds(c4 * 16, 16)
                acc = bias_v[sl]
                for j in range(K):
                    acc = acc + rows_v[i * K + j, sl]
                out_v[i, sl] = acc
            return c2

        lax.fori_loop(0, CH, pos_body, 0)
        pltpu.sync_copy(out_v, out_hbm.at[pl.ds(base, CH)])
        return carry

    lax.fori_loop(0, NCHUNK, chunk_body, 0)


@functools.cache
def _gather_sum():
    mesh = plsc.VectorSubcoreMesh(core_axis_name="c", subcore_axis_name="s")
    return pl.kernel(
        _gather_sum_body,
        mesh=mesh,
        out_type=jax.ShapeDtypeStruct((POS, DK), jnp.float32),
        scratch_types=[
            pltpu.VMEM((K, CH), jnp.int32),
            pltpu.VMEM((K * CH, DK), jnp.float32),
            pltpu.VMEM((CH, DK), jnp.float32),
            pltpu.VMEM((DK,), jnp.float32),
            pltpu.SemaphoreType.DMA,
        ],
        compiler_params=pltpu.CompilerParams(use_tc_tiling_on_sc=False),
    )


def kernel(x, conv_w, conv_b):
    xh = x.reshape(T, H, DK).transpose(1, 0, 2)
    wt = conv_w.transpose(2, 1, 0)
    fidx, y = _sim_topk(xh, wt)
    table = y.reshape(POS * K, DK)
    fidx_r = fidx.reshape(NW * NCHUNK, K, CH)
    out_sc = _gather_sum()(table, fidx_r, conv_b)
    y2 = out_sc.reshape(1, H, DK, T)
    return y2.transpose(0, 1, 3, 2).transpose(0, 2, 1, 3).reshape(1, T, H * DK)


# speedup vs baseline: 28.6411x; 1.0260x over previous
import functools

import jax
import jax.numpy as jnp
from jax import lax
from jax.experimental import pallas as pl
from jax.experimental.pallas import tpu as pltpu
from jax.experimental.pallas import tpu_sc as plsc

H = 12
T = 2048
DK = 64
K = 4
TM = 256
NB = T // TM

NC = 2
NS = 16
NW = NC * NS
POS = H * T
PPW = POS // NW
CH = 128
NCHUNK = PPW // CH


def _sim_topk_body(xfull_ref, xblk_ref, wt_ref, fidx_ref, y_ref, xn_scr):
    h = pl.program_id(0)
    i = pl.program_id(1)
    xblk = xblk_ref[0]

    @pl.when(i == 0)
    def _():
        xfull = xfull_ref[0]
        nf = jnp.sqrt(jnp.sum(xfull * xfull, axis=1, keepdims=True))
        xn_scr[...] = xfull / jnp.maximum(nf, 1e-12)

    xn_full = xn_scr[...]
    xn_blk = xn_scr[pl.ds(i * TM, TM), :]
    s = lax.dot_general(xn_blk, xn_full, (((1,), (1,)), ((), ())),
                        preferred_element_type=jnp.float32)
    s = jnp.maximum(s, 0.0)
    iota = lax.broadcasted_iota(jnp.int32, (TM, T), 1)
    cols = []
    for _ in range(K):
        m = jnp.max(s, axis=1, keepdims=True)
        ii = jnp.min(jnp.where(s == m, iota, T), axis=1, keepdims=True)
        cols.append(ii)
        s = jnp.where(iota == ii, -1.0, s)
    idx = jnp.concatenate(cols, axis=1)
    jj = lax.broadcasted_iota(jnp.int32, (TM, K), 1)
    fidx_ref[0] = (h * T + idx) * K + jj
    ys = [lax.dot_general(xblk, wt_ref[j], (((1,), (0,)), ((), ())),
                          preferred_element_type=jnp.float32)
          for j in range(K)]
    y_ref[0] = jnp.concatenate(ys, axis=1)


def _sim_topk(xh, wt):
    return pl.pallas_call(
        _sim_topk_body,
        grid=(H, NB),
        in_specs=[
            pl.BlockSpec((1, T, DK), lambda h, i: (h, 0, 0)),
            pl.BlockSpec((1, TM, DK), lambda h, i: (h, i, 0)),
            pl.BlockSpec((K, DK, DK), lambda h, i: (0, 0, 0)),
        ],
        out_specs=[
            pl.BlockSpec((1, TM, K), lambda h, i: (h, i, 0)),
            pl.BlockSpec((1, TM, K * DK), lambda h, i: (h, i, 0)),
        ],
        out_shape=[
            jax.ShapeDtypeStruct((H, T, K), jnp.int32),
            jax.ShapeDtypeStruct((H, T, K * DK), jnp.float32),
        ],
        scratch_shapes=[pltpu.VMEM((T, DK), jnp.float32)],
    )(xh, xh, wt)


def _gather_sum_body(table_hbm, fidx_hbm, bias_hbm, out_hbm,
                     idx_v, rows_v, out_v, bias_v, gsem0, gsem1, osem):
    cid = lax.axis_index("c")
    sid = lax.axis_index("s")
    wid = sid * NC + cid
    pltpu.sync_copy(bias_hbm, bias_v)
    gsems = [gsem0, gsem1]
    gcopies = [None, None]
    ocopies = [None, None]

    def fire(c):
        b = c % 2
        gchunk = wid * NCHUNK + c
        pltpu.sync_copy(fidx_hbm.at[gchunk], idx_v.at[b])
        gcopies[b] = [
            pltpu.async_copy(table_hbm.at[idx_v.at[b, g]],
                             rows_v.at[b, pl.ds(g * CH, CH)], gsems[b])
            for g in range(K)
        ]

    fire(0)
    for c in range(NCHUNK):
        if c + 1 < NCHUNK:
            fire(c + 1)
        b = c % 2
        for cp in gcopies[b]:
            cp.wait()
        if ocopies[b] is not None:
            ocopies[b].wait()

        def pos_body(i, c2, b=b):
            for c4 in range(DK // 16):
                sl = pl.ds(c4 * 16, 16)
                acc = bias_v[sl]
                for j in range(K):
                    acc = acc + rows_v[b, i * K + j, sl]
                out_v[b, i, sl] = acc
            return c2

        lax.fori_loop(0, CH, pos_body, 0)
        base = (wid * NCHUNK + c) * CH
        ocopies[b] = pltpu.async_copy(out_v.at[b], out_hbm.at[pl.ds(base, CH)],
                                      osem)
    for oc in ocopies:
        if oc is not None:
            oc.wait()


@functools.cache
def _gather_sum():
    mesh = plsc.VectorSubcoreMesh(core_axis_name="c", subcore_axis_name="s")
    return pl.kernel(
        _gather_sum_body,
        mesh=mesh,
        out_type=jax.ShapeDtypeStruct((POS, DK), jnp.float32),
        scratch_types=[
            pltpu.VMEM((2, K, CH), jnp.int32),
            pltpu.VMEM((2, K * CH, DK), jnp.float32),
            pltpu.VMEM((2, CH, DK), jnp.float32),
            pltpu.VMEM((DK,), jnp.float32),
            pltpu.SemaphoreType.DMA,
            pltpu.SemaphoreType.DMA,
            pltpu.SemaphoreType.DMA,
        ],
        compiler_params=pltpu.CompilerParams(use_tc_tiling_on_sc=False),
    )


def kernel(x, conv_w, conv_b):
    xh = x.reshape(T, H, DK).transpose(1, 0, 2)
    wt = conv_w.transpose(2, 1, 0)
    fidx, y = _sim_topk(xh, wt)
    table = y.reshape(POS * K, DK)
    fidx_r = fidx.reshape(NW * NCHUNK, K, CH)
    out_sc = _gather_sum()(table, fidx_r, conv_b)
    y2 = out_sc.reshape(1, H, DK, T)
    return y2.transpose(0, 1, 3, 2).transpose(0, 2, 1, 3).reshape(1, T, H * DK)
